# Initial kernel scaffold; baseline (speedup 1.0000x reference)
import jax
import jax.numpy as jnp
from jax.experimental import pallas as pl


def kernel(x, edge_index, W1, b1, g1, be1, a1, W2, b2, g2, be2, a2):
    raise NotImplementedError("write your pallas kernel here")


# speedup vs baseline: 15.9696x; 15.9696x over previous
import functools

import jax
import jax.numpy as jnp
from jax import lax
from jax.experimental import pallas as pl
from jax.experimental.pallas import tpu as pltpu
from jax.experimental.pallas import tpu_sc as plsc

N = 10000
E = 320000
D = 128

NC = 2
NS = 16
NW = NC * NS
EPW = E // NW
C = 128
NFULL = EPW // C
TAIL = EPW - NFULL * C
NP = 10240
RPT = NP // NS

_mesh = plsc.VectorSubcoreMesh(core_axis_name="c", subcore_axis_name="s")
_sc_params = pltpu.CompilerParams(use_tc_tiling_on_sc=False)


def _deg_sc_body(dst_hbm, out_hbm, dstv, dstv_t, ones_v, ones_t, zbuf, zbuf1, acc):
    c = lax.axis_index("c")
    s = lax.axis_index("s")

    def fill(i, _):
        zbuf[i, :] = jnp.zeros((16,), jnp.float32)
        ones_v[i % C, :] = jnp.ones((16,), jnp.float32)
        ones_t[i % TAIL, :] = jnp.ones((16,), jnp.float32)
        return 0

    lax.fori_loop(0, RPT, fill, 0)
    pltpu.sync_copy(zbuf, acc.at[pl.ds(s * RPT, RPT)])
    plsc.subcore_barrier()

    base0 = c * (E // NC) + s * EPW

    def body(i, _):
        pltpu.sync_copy(dst_hbm.at[pl.ds(base0 + i * C, C)], dstv)
        pltpu.sync_copy(ones_v, acc.at[dstv], add=True)
        return 0

    lax.fori_loop(0, NFULL, body, 0)
    pltpu.sync_copy(dst_hbm.at[pl.ds(base0 + NFULL * C, TAIL)], dstv_t)
    pltpu.sync_copy(ones_t, acc.at[dstv_t], add=True)
    plsc.subcore_barrier()
    pltpu.sync_copy(acc.at[pl.ds(s * RPT, RPT)], zbuf)

    def repack(i, _):
        zbuf1[pl.ds(i * 16, 16)] = zbuf[i, :]
        return 0

    lax.fori_loop(0, RPT, repack, 0)
    pltpu.sync_copy(zbuf1, out_hbm.at[pl.ds((c * NP + s * RPT) * 16, RPT * 16)])


def _agg_sc_body(h_hbm, src_hbm, dst_hbm, out_hbm,
            srcv, dstv, srcv_t, dstv_t, rows, rows_t, zbuf, acc, sem):
    c = lax.axis_index("c")
    s = lax.axis_index("s")

    def fill(i, _):
        q, r = i // 8, i % 8
        zbuf[q, pl.ds(r * 16, 16)] = jnp.zeros((16,), jnp.float32)
        return 0

    lax.fori_loop(0, (RPT // 5) * 8, fill, 0)
    for k in range(5):
        pltpu.sync_copy(zbuf, acc.at[pl.ds(s * RPT + k * (RPT // 5), RPT // 5)])
    plsc.subcore_barrier()

    base0 = c * (E // NC) + s * EPW

    def body(i, _):
        pltpu.sync_copy(src_hbm.at[pl.ds(base0 + i * C, C)], srcv)
        pltpu.sync_copy(dst_hbm.at[pl.ds(base0 + i * C, C)], dstv)
        pltpu.async_copy(h_hbm.at[srcv], rows, sem).wait()
        pltpu.sync_copy(rows, acc.at[dstv], add=True)
        return 0

    lax.fori_loop(0, NFULL, body, 0)
    tbase = base0 + NFULL * C
    pltpu.sync_copy(src_hbm.at[pl.ds(tbase, TAIL)], srcv_t)
    pltpu.sync_copy(dst_hbm.at[pl.ds(tbase, TAIL)], dstv_t)
    pltpu.async_copy(h_hbm.at[srcv_t], rows_t, sem).wait()
    pltpu.sync_copy(rows_t, acc.at[dstv_t], add=True)
    plsc.subcore_barrier()
    for k in range(5):
        off = s * RPT + k * (RPT // 5)
        pltpu.sync_copy(acc.at[pl.ds(off, RPT // 5)], zbuf)
        pltpu.sync_copy(zbuf, out_hbm.at[pl.ds(c * NP + off, RPT // 5)])


_deg_sc = pl.kernel(
    _deg_sc_body,
    out_type=jax.ShapeDtypeStruct((NC * NP * 16,), jnp.float32),
    mesh=_mesh,
    scratch_types=[
        pltpu.VMEM((C,), jnp.int32),
        pltpu.VMEM((TAIL,), jnp.int32),
        pltpu.VMEM((C, 16), jnp.float32),
        pltpu.VMEM((TAIL, 16), jnp.float32),
        pltpu.VMEM((RPT, 16), jnp.float32),
        pltpu.VMEM((RPT * 16,), jnp.float32),
        pltpu.VMEM_SHARED((NP, 16), jnp.float32),
    ],
    compiler_params=_sc_params,
)

_agg_sc = pl.kernel(
    _agg_sc_body,
    out_type=jax.ShapeDtypeStruct((NC * NP, D), jnp.float32),
    mesh=_mesh,
    scratch_types=[
        pltpu.VMEM((C,), jnp.int32),
        pltpu.VMEM((C,), jnp.int32),
        pltpu.VMEM((TAIL,), jnp.int32),
        pltpu.VMEM((TAIL,), jnp.int32),
        pltpu.VMEM((C, D), jnp.float32),
        pltpu.VMEM((TAIL, D), jnp.float32),
        pltpu.VMEM((RPT // 5, D), jnp.float32),
        pltpu.VMEM_SHARED((NP, D), jnp.float32),
        pltpu.SemaphoreType.DMA,
    ],
    compiler_params=_sc_params,
)


R = 1000


def _dinv_of(p_ref):
    deg = 1.0 + p_ref[0, :, 0:1] + p_ref[1, :, 0:1]
    return lax.rsqrt(deg)


def _tc1_body(p_ref, x_ref, w_ref, o_ref):
    dinv = _dinv_of(p_ref)
    h = jnp.dot(x_ref[...], w_ref[...], preferred_element_type=jnp.float32)
    o_ref[...] = h * dinv


def _ln_prelu(pre, g_ref, be_ref, a_ref):
    mu = jnp.mean(pre, axis=-1, keepdims=True)
    dm = pre - mu
    var = jnp.mean(dm * dm, axis=-1, keepdims=True)
    y = dm * lax.rsqrt(var + 1e-5) * g_ref[...] + be_ref[...]
    a = a_ref[0]
    return jnp.where(y >= 0, y, a * y)


def _tc2_body(p_ref, s_ref, hs_ref, b_ref, g_ref, be_ref, a_ref, w_ref, o_ref):
    dinv = _dinv_of(p_ref)
    pre = (s_ref[0] + s_ref[1] + hs_ref[...]) * dinv + b_ref[...]
    y = _ln_prelu(pre, g_ref, be_ref, a_ref)
    h = jnp.dot(y, w_ref[...], preferred_element_type=jnp.float32)
    o_ref[...] = h * dinv


def _tc3_body(p_ref, s_ref, hs_ref, b_ref, g_ref, be_ref, a_ref, o_ref):
    dinv = _dinv_of(p_ref)
    pre = (s_ref[0] + s_ref[1] + hs_ref[...]) * dinv + b_ref[...]
    o_ref[...] = _ln_prelu(pre, g_ref, be_ref, a_ref)


_p_spec = pl.BlockSpec((2, R, 16), lambda i: (0, i, 0))
_s_spec = pl.BlockSpec((2, R, D), lambda i: (0, i, 0))
_row_spec = pl.BlockSpec((R, D), lambda i: (i, 0))
_w_spec = pl.BlockSpec((D, D), lambda i: (0, 0))
_v_spec = pl.BlockSpec((1, D), lambda i: (0, 0))
_a_spec = pl.BlockSpec(memory_space=pltpu.SMEM)
_out_shape = jax.ShapeDtypeStruct((N, D), jnp.float32)

_tc1 = pl.pallas_call(
    _tc1_body, grid=(N // R,),
    in_specs=[_p_spec, _row_spec, _w_spec],
    out_specs=_row_spec, out_shape=_out_shape)

_tc2 = pl.pallas_call(
    _tc2_body, grid=(N // R,),
    in_specs=[_p_spec, _s_spec, _row_spec, _v_spec, _v_spec, _v_spec, _a_spec,
              _w_spec],
    out_specs=_row_spec, out_shape=_out_shape)

_tc3 = pl.pallas_call(
    _tc3_body, grid=(N // R,),
    in_specs=[_p_spec, _s_spec, _row_spec, _v_spec, _v_spec, _v_spec, _a_spec],
    out_specs=_row_spec, out_shape=_out_shape)


def kernel(x, edge_index, W1, b1, g1, be1, a1, W2, b2, g2, be2, a2):
    src = edge_index[0]
    dst = edge_index[1]
    p = _deg_sc(dst).reshape(NC, NP, 16)
    h1s = _tc1(p, x, W1)
    s1 = _agg_sc(h1s, src, dst).reshape(NC, NP, D)
    h2s = _tc2(p, s1, h1s, b1.reshape(1, D), g1.reshape(1, D),
               be1.reshape(1, D), a1, W2)
    s2 = _agg_sc(h2s, src, dst).reshape(NC, NP, D)
    out = _tc3(p, s2, h2s, b2.reshape(1, D), g2.reshape(1, D),
               be2.reshape(1, D), a2)
    return out


# speedup vs baseline: 23.0196x; 1.4415x over previous
import functools

import jax
import jax.numpy as jnp
from jax import lax
from jax.experimental import pallas as pl
from jax.experimental.pallas import tpu as pltpu
from jax.experimental.pallas import tpu_sc as plsc

N = 10000
E = 320000
D = 128

NC = 2
NS = 16
NW = NC * NS
EPW = E // NW
C = 128
NFULL = EPW // C
TAIL = EPW - NFULL * C
NP = 10240
RPT = NP // NS

_mesh = plsc.VectorSubcoreMesh(core_axis_name="c", subcore_axis_name="s")
_sc_params = pltpu.CompilerParams(use_tc_tiling_on_sc=False)


def _deg_sc_body(dst_hbm, out_hbm, dstv, dstv_t, ones_v, ones_t, zbuf, zbuf1, acc):
    c = lax.axis_index("c")
    s = lax.axis_index("s")

    def fill(i, _):
        zbuf[i, :] = jnp.zeros((16,), jnp.float32)
        ones_v[i % C, :] = jnp.ones((16,), jnp.float32)
        ones_t[i % TAIL, :] = jnp.ones((16,), jnp.float32)
        return 0

    lax.fori_loop(0, RPT, fill, 0)
    pltpu.sync_copy(zbuf, acc.at[pl.ds(s * RPT, RPT)])
    plsc.subcore_barrier()

    base0 = c * (E // NC) + s * EPW

    def body(i, _):
        pltpu.sync_copy(dst_hbm.at[pl.ds(base0 + i * C, C)], dstv)
        pltpu.sync_copy(ones_v, acc.at[dstv], add=True)
        return 0

    lax.fori_loop(0, NFULL, body, 0)
    pltpu.sync_copy(dst_hbm.at[pl.ds(base0 + NFULL * C, TAIL)], dstv_t)
    pltpu.sync_copy(ones_t, acc.at[dstv_t], add=True)
    plsc.subcore_barrier()
    pltpu.sync_copy(acc.at[pl.ds(s * RPT, RPT)], zbuf)

    def repack(i, _):
        zbuf1[pl.ds(i * 16, 16)] = zbuf[i, :]
        return 0

    lax.fori_loop(0, RPT, repack, 0)
    pltpu.sync_copy(zbuf1, out_hbm.at[pl.ds((c * NP + s * RPT) * 16, RPT * 16)])


ZR = 32


def _agg_sc_body(h_hbm, src_hbm, dst_hbm, out_hbm,
                 srcv0, dstv0, srcv1, dstv1, srcv_t, dstv_t,
                 rows0, rows1, rows_t, zbuf, acc, g0, g1, s0, s1):
    c = lax.axis_index("c")
    s = lax.axis_index("s")

    def fill(i, _):
        q, r = i // 8, i % 8
        zbuf[q, pl.ds(r * 16, 16)] = jnp.zeros((16,), jnp.float32)
        return 0

    lax.fori_loop(0, ZR * 8, fill, 0)
    for k in range(RPT // ZR):
        pltpu.sync_copy(zbuf, acc.at[pl.ds(s * RPT + k * ZR, ZR)])
    plsc.subcore_barrier()

    base0 = c * (E // NC) + s * EPW
    pltpu.sync_copy(src_hbm.at[pl.ds(base0, C)], srcv0)
    pltpu.sync_copy(dst_hbm.at[pl.ds(base0, C)], dstv0)
    ga = pltpu.async_copy(h_hbm.at[srcv0], rows0, g0)

    def body(j, _):
        b1 = base0 + (2 * j + 1) * C
        b2 = base0 + (2 * j + 2) * C
        pltpu.sync_copy(src_hbm.at[pl.ds(b1, C)], srcv1)
        pltpu.sync_copy(dst_hbm.at[pl.ds(b1, C)], dstv1)
        gb = pltpu.async_copy(h_hbm.at[srcv1], rows1, g1)
        pltpu.make_async_copy(h_hbm.at[srcv0], rows0, g0).wait()
        sa = pltpu.async_copy(rows0, acc.at[dstv0], s0, add=True)
        sa.wait()
        pltpu.sync_copy(src_hbm.at[pl.ds(b2, C)], srcv0)
        pltpu.sync_copy(dst_hbm.at[pl.ds(b2, C)], dstv0)
        pltpu.async_copy(h_hbm.at[srcv0], rows0, g0)
        gb.wait()
        sb = pltpu.async_copy(rows1, acc.at[dstv1], s1, add=True)
        sb.wait()
        return 0

    lax.fori_loop(0, NFULL // 2 - 1, body, 0)

    bl = base0 + (NFULL - 1) * C
    pltpu.sync_copy(src_hbm.at[pl.ds(bl, C)], srcv1)
    pltpu.sync_copy(dst_hbm.at[pl.ds(bl, C)], dstv1)
    gb = pltpu.async_copy(h_hbm.at[srcv1], rows1, g1)
    pltpu.make_async_copy(h_hbm.at[srcv0], rows0, g0).wait()
    sa = pltpu.async_copy(rows0, acc.at[dstv0], s0, add=True)
    tbase = base0 + NFULL * C
    pltpu.sync_copy(src_hbm.at[pl.ds(tbase, TAIL)], srcv_t)
    pltpu.sync_copy(dst_hbm.at[pl.ds(tbase, TAIL)], dstv_t)
    gt = pltpu.async_copy(h_hbm.at[srcv_t], rows_t, g0)
    gb.wait()
    sb = pltpu.async_copy(rows1, acc.at[dstv1], s1, add=True)
    gt.wait()
    sa.wait()
    sb.wait()
    pltpu.sync_copy(rows_t, acc.at[dstv_t], add=True)
    plsc.subcore_barrier()
    for k in range(RPT // ZR):
        off = s * RPT + k * ZR
        pltpu.sync_copy(acc.at[pl.ds(off, ZR)], zbuf)
        pltpu.sync_copy(zbuf, out_hbm.at[pl.ds(c * NP + off, ZR)])


_deg_sc = pl.kernel(
    _deg_sc_body,
    out_type=jax.ShapeDtypeStruct((NC * NP * 16,), jnp.float32),
    mesh=_mesh,
    scratch_types=[
        pltpu.VMEM((C,), jnp.int32),
        pltpu.VMEM((TAIL,), jnp.int32),
        pltpu.VMEM((C, 16), jnp.float32),
        pltpu.VMEM((TAIL, 16), jnp.float32),
        pltpu.VMEM((RPT, 16), jnp.float32),
        pltpu.VMEM((RPT * 16,), jnp.float32),
        pltpu.VMEM_SHARED((NP, 16), jnp.float32),
    ],
    compiler_params=_sc_params,
)

_agg_sc = pl.kernel(
    _agg_sc_body,
    out_type=jax.ShapeDtypeStruct((NC * NP, D), jnp.float32),
    mesh=_mesh,
    scratch_types=[
        pltpu.VMEM((C,), jnp.int32),
        pltpu.VMEM((C,), jnp.int32),
        pltpu.VMEM((C,), jnp.int32),
        pltpu.VMEM((C,), jnp.int32),
        pltpu.VMEM((TAIL,), jnp.int32),
        pltpu.VMEM((TAIL,), jnp.int32),
        pltpu.VMEM((C, D), jnp.float32),
        pltpu.VMEM((C, D), jnp.float32),
        pltpu.VMEM((TAIL, D), jnp.float32),
        pltpu.VMEM((ZR, D), jnp.float32),
        pltpu.VMEM_SHARED((NP, D), jnp.float32),
        pltpu.SemaphoreType.DMA,
        pltpu.SemaphoreType.DMA,
        pltpu.SemaphoreType.DMA,
        pltpu.SemaphoreType.DMA,
    ],
    compiler_params=_sc_params,
)


R = 1000


def _dinv_of(p_ref):
    deg = 1.0 + p_ref[0, :, 0:1] + p_ref[1, :, 0:1]
    return lax.rsqrt(deg)


def _tc1_body(p_ref, x_ref, w_ref, o_ref):
    dinv = _dinv_of(p_ref)
    h = jnp.dot(x_ref[...], w_ref[...], preferred_element_type=jnp.float32)
    o_ref[...] = h * dinv


def _ln_prelu(pre, g_ref, be_ref, a_ref):
    mu = jnp.mean(pre, axis=-1, keepdims=True)
    dm = pre - mu
    var = jnp.mean(dm * dm, axis=-1, keepdims=True)
    y = dm * lax.rsqrt(var + 1e-5) * g_ref[...] + be_ref[...]
    a = a_ref[0]
    return jnp.where(y >= 0, y, a * y)


def _tc2_body(p_ref, s_ref, hs_ref, b_ref, g_ref, be_ref, a_ref, w_ref, o_ref):
    dinv = _dinv_of(p_ref)
    pre = (s_ref[0] + s_ref[1] + hs_ref[...]) * dinv + b_ref[...]
    y = _ln_prelu(pre, g_ref, be_ref, a_ref)
    h = jnp.dot(y, w_ref[...], preferred_element_type=jnp.float32)
    o_ref[...] = h * dinv


def _tc3_body(p_ref, s_ref, hs_ref, b_ref, g_ref, be_ref, a_ref, o_ref):
    dinv = _dinv_of(p_ref)
    pre = (s_ref[0] + s_ref[1] + hs_ref[...]) * dinv + b_ref[...]
    o_ref[...] = _ln_prelu(pre, g_ref, be_ref, a_ref)


_p_spec = pl.BlockSpec((2, R, 16), lambda i: (0, i, 0))
_s_spec = pl.BlockSpec((2, R, D), lambda i: (0, i, 0))
_row_spec = pl.BlockSpec((R, D), lambda i: (i, 0))
_w_spec = pl.BlockSpec((D, D), lambda i: (0, 0))
_v_spec = pl.BlockSpec((1, D), lambda i: (0, 0))
_a_spec = pl.BlockSpec(memory_space=pltpu.SMEM)
_out_shape = jax.ShapeDtypeStruct((N, D), jnp.float32)

_tc1 = pl.pallas_call(
    _tc1_body, grid=(N // R,),
    in_specs=[_p_spec, _row_spec, _w_spec],
    out_specs=_row_spec, out_shape=_out_shape)

_tc2 = pl.pallas_call(
    _tc2_body, grid=(N // R,),
    in_specs=[_p_spec, _s_spec, _row_spec, _v_spec, _v_spec, _v_spec, _a_spec,
              _w_spec],
    out_specs=_row_spec, out_shape=_out_shape)

_tc3 = pl.pallas_call(
    _tc3_body, grid=(N // R,),
    in_specs=[_p_spec, _s_spec, _row_spec, _v_spec, _v_spec, _v_spec, _a_spec],
    out_specs=_row_spec, out_shape=_out_shape)


def kernel(x, edge_index, W1, b1, g1, be1, a1, W2, b2, g2, be2, a2):
    src = edge_index[0]
    dst = edge_index[1]
    p = _deg_sc(dst).reshape(NC, NP, 16)
    h1s = _tc1(p, x, W1)
    s1 = _agg_sc(h1s, src, dst).reshape(NC, NP, D)
    h2s = _tc2(p, s1, h1s, b1.reshape(1, D), g1.reshape(1, D),
               be1.reshape(1, D), a1, W2)
    s2 = _agg_sc(h2s, src, dst).reshape(NC, NP, D)
    out = _tc3(p, s2, h2s, b2.reshape(1, D), g2.reshape(1, D),
               be2.reshape(1, D), a2)
    return out
